# baseline (speedup 1.0000x reference)
import functools

import jax
import jax.numpy as jnp
from jax import lax
from jax.experimental import pallas as pl
from jax.experimental.pallas import tpu as pltpu
from jax.experimental.pallas import tpu_sc as plsc

NC = 2
NS = 16
L = 16
NW = NC * NS

B = 16384
F = 16
BPW = B // NW
CH = 128
NCH = BPW // CH

_mesh = plsc.VectorSubcoreMesh(core_axis_name="c", subcore_axis_name="s")


@functools.partial(
    pl.kernel,
    out_type=jax.ShapeDtypeStruct((B,), jnp.float32),
    mesh=_mesh,
    compiler_params=pltpu.CompilerParams(
        needs_layout_passes=False, use_tc_tiling_on_sc=False),
    scratch_types=[
        pltpu.VMEM((NCH, CH), jnp.int32),
        pltpu.VMEM((NCH, CH), jnp.int32),
        pltpu.VMEM((BPW, F), jnp.float32),
        pltpu.VMEM((BPW, F), jnp.float32),
        pltpu.VMEM((BPW,), jnp.float32),
        pltpu.SemaphoreType.DMA,
    ],
)
def _mf_score(task_hbm, worker_hbm, tf_hbm, wf_hbm, out_hbm,
              t_idx, w_idx, t_rows, w_rows, out_v, sem):
    wid = lax.axis_index("s") * NC + lax.axis_index("c")
    base = wid * BPW

    for j in range(NCH):
        pltpu.sync_copy(task_hbm.at[pl.ds(base + j * CH, CH)], t_idx.at[j])
        pltpu.sync_copy(worker_hbm.at[pl.ds(base + j * CH, CH)], w_idx.at[j])

    copies = []
    for j in range(NCH):
        copies.append(pltpu.async_copy(
            tf_hbm.at[t_idx.at[j]], t_rows.at[pl.ds(j * CH, CH)], sem))
        copies.append(pltpu.async_copy(
            wf_hbm.at[w_idx.at[j]], w_rows.at[pl.ds(j * CH, CH)], sem))
    for cp in copies:
        cp.wait()

    lanes = lax.iota(jnp.int32, L)

    def chunk(c, carry):
        rows = c * L + lanes
        acc = jnp.zeros((L,), jnp.float32)
        for f in range(F):
            col = jnp.full((L,), f, jnp.int32)
            tv = plsc.load_gather(t_rows, [rows, col])
            wv = plsc.load_gather(w_rows, [rows, col])
            acc = acc + tv * wv
        out_v[pl.ds(c * L, L)] = 1.0 / (1.0 + jnp.exp(-acc))
        return carry

    lax.fori_loop(0, BPW // L, chunk, 0)

    pltpu.sync_copy(out_v, out_hbm.at[pl.ds(base, BPW)])


def kernel(task, worker, task_factors, worker_factors):
    return _mf_score(task.astype(jnp.int32), worker.astype(jnp.int32),
                     task_factors, worker_factors)


# speedup vs baseline: 1.0017x; 1.0017x over previous
import functools

import jax
import jax.numpy as jnp
from jax import lax
from jax.experimental import pallas as pl
from jax.experimental.pallas import tpu as pltpu
from jax.experimental.pallas import tpu_sc as plsc

NC = 2
NS = 16
L = 16
NW = NC * NS

B = 16384
F = 16
RPT = 8
BPW = B // NW
CH = 128
NCH = BPW // CH
NBUF = 2

_mesh = plsc.VectorSubcoreMesh(core_axis_name="c", subcore_axis_name="s")


@functools.partial(
    pl.kernel,
    out_type=jax.ShapeDtypeStruct((B,), jnp.float32),
    mesh=_mesh,
    compiler_params=pltpu.CompilerParams(needs_layout_passes=False),
    scratch_types=[
        pltpu.VMEM((BPW,), jnp.int32),
        pltpu.VMEM((BPW,), jnp.int32),
        pltpu.VMEM((NCH, CH), jnp.int32),
        pltpu.VMEM((NCH, CH), jnp.int32),
        pltpu.VMEM((NBUF, CH, RPT * F), jnp.float32),
        pltpu.VMEM((NBUF, CH, RPT * F), jnp.float32),
        pltpu.VMEM((BPW,), jnp.float32),
        pltpu.SemaphoreType.DMA,
        pltpu.SemaphoreType.DMA,
    ],
)
def _mf_score(task_hbm, worker_hbm, tf_hbm, wf_hbm, out_hbm,
              t_idx, w_idx, t_row, w_row, t_buf, w_buf, out_v,
              sem_idx, sem):
    wid = lax.axis_index("s") * NC + lax.axis_index("c")
    base = wid * BPW

    cp_t = pltpu.async_copy(task_hbm.at[pl.ds(base, BPW)], t_idx, sem_idx)
    cp_w = pltpu.async_copy(worker_hbm.at[pl.ds(base, BPW)], w_idx, sem_idx)
    cp_t.wait()
    cp_w.wait()

    for j in range(NCH):
        for k in range(CH // L):
            s = pl.ds(j * CH + k * L, L)
            d = pl.ds(k * L, L)
            t_row[j, d] = lax.shift_right_logical(t_idx[s], 3)
            w_row[j, d] = lax.shift_right_logical(w_idx[s], 3)

    def fire(j):
        b = j % NBUF
        return (pltpu.async_copy(tf_hbm.at[t_row.at[j]], t_buf.at[b], sem),
                pltpu.async_copy(wf_hbm.at[w_row.at[j]], w_buf.at[b], sem))

    lanes = lax.iota(jnp.int32, L)
    pending = fire(0)

    for j in range(NCH):
        nxt = fire(j + 1) if j + 1 < NCH else None
        pending[0].wait()
        pending[1].wait()
        b = j % NBUF

        def group(g, carry, b=b, j=j):
            rows = g * L + lanes
            ti = t_idx[pl.ds(j * CH + g * L, L)]
            wi = w_idx[pl.ds(j * CH + g * L, L)]
            tcol = (ti & (RPT - 1)) * F
            wcol = (wi & (RPT - 1)) * F
            acc = jnp.zeros((L,), jnp.float32)
            for f in range(F):
                tv = plsc.load_gather(t_buf.at[b], [rows, tcol + f])
                wv = plsc.load_gather(w_buf.at[b], [rows, wcol + f])
                acc = acc + tv * wv
            out_v[pl.ds(j * CH + g * L, L)] = 1.0 / (1.0 + jnp.exp(-acc))
            return carry

        lax.fori_loop(0, CH // L, group, 0)
        pending = nxt

    pltpu.sync_copy(out_v, out_hbm.at[pl.ds(base, BPW)])


def kernel(task, worker, task_factors, worker_factors):
    tf2 = task_factors.reshape(-1, RPT * F)
    wf2 = worker_factors.reshape(-1, RPT * F)
    return _mf_score(task.astype(jnp.int32), worker.astype(jnp.int32),
                     tf2, wf2)


# speedup vs baseline: 5.1657x; 5.1568x over previous
import functools

import jax
import jax.numpy as jnp
from jax import lax
from jax.experimental import pallas as pl
from jax.experimental.pallas import tpu as pltpu
from jax.experimental.pallas import tpu_sc as plsc

NC = 2
NS = 16
L = 16
NW = NC * NS

B = 16384
F = 16
BPW = B // NW
G = 16
NSTEP = (BPW // G) * 2

_mesh = plsc.VectorSubcoreMesh(core_axis_name="c", subcore_axis_name="s")


@functools.partial(
    pl.kernel,
    out_type=jax.ShapeDtypeStruct((B,), jnp.float32),
    mesh=_mesh,
    compiler_params=pltpu.CompilerParams(needs_layout_passes=False),
    scratch_types=[
        pltpu.VMEM((BPW,), jnp.int32),
        pltpu.VMEM((BPW,), jnp.int32),
        pltpu.VMEM((G, 8, 128), jnp.float32),
        pltpu.VMEM((G, 8, 128), jnp.float32),
        pltpu.VMEM((BPW,), jnp.float32),
        pltpu.SemaphoreType.DMA,
        pltpu.SemaphoreType.DMA,
    ],
)
def _mf_score(task_hbm, worker_hbm, tfT_hbm, wfT_hbm, out_hbm,
              t_idx, w_idx, t_slab, w_slab, out_v, sem_idx, sem):
    wid = lax.axis_index("s") * NC + lax.axis_index("c")
    base = wid * BPW

    cp_t = pltpu.async_copy(task_hbm.at[pl.ds(base, BPW)], t_idx, sem_idx)
    cp_w = pltpu.async_copy(worker_hbm.at[pl.ds(base, BPW)], w_idx, sem_idx)
    cp_t.wait()
    cp_w.wait()

    lanes = lax.iota(jnp.int32, L)

    def step(s, acc):
        g = s >> 1
        p = s & 1
        off = pl.multiple_of(g * G, G)
        ivt = t_idx[pl.ds(off, G)]
        ivw = w_idx[pl.ds(off, G)]
        rb = pl.multiple_of(p * 8, 8)
        cps = []
        for k in range(G):
            ct = pl.multiple_of(ivt[k] & -128, 128)
            cw = pl.multiple_of(ivw[k] & -128, 128)
            cps.append(pltpu.async_copy(
                tfT_hbm.at[pl.ds(rb, 8), pl.ds(ct, 128)], t_slab.at[k], sem))
            cps.append(pltpu.async_copy(
                wfT_hbm.at[pl.ds(rb, 8), pl.ds(cw, 128)], w_slab.at[k], sem))
        for cp in cps:
            cp.wait()

        lt = ivt & 127
        lw = ivw & 127
        c = jnp.zeros((L,), jnp.float32)
        for f in range(8):
            fv = jnp.full((L,), f, jnp.int32)
            tv = plsc.load_gather(t_slab, [lanes, fv, lt])
            wv = plsc.load_gather(w_slab, [lanes, fv, lw])
            c = c + tv * wv
        acc = acc + c

        @pl.when(p == 1)
        def _():
            out_v[pl.ds(off, G)] = 1.0 / (1.0 + jnp.exp(-acc))

        return jnp.where(p == 1, 0.0, acc)

    lax.fori_loop(0, NSTEP, step, jnp.zeros((L,), jnp.float32))

    pltpu.sync_copy(out_v, out_hbm.at[pl.ds(base, BPW)])


def kernel(task, worker, task_factors, worker_factors):
    return _mf_score(task.astype(jnp.int32), worker.astype(jnp.int32),
                     task_factors.T, worker_factors.T)


# speedup vs baseline: 6.1243x; 1.1856x over previous
import functools

import jax
import jax.numpy as jnp
from jax import lax
from jax.experimental import pallas as pl
from jax.experimental.pallas import tpu as pltpu
from jax.experimental.pallas import tpu_sc as plsc

NC = 2
NS = 16
L = 16
NW = NC * NS

B = 16384
F = 16
BPW = B // NW
G = 16
NSTEP = BPW // G

_mesh = plsc.VectorSubcoreMesh(core_axis_name="c", subcore_axis_name="s")


@functools.partial(
    pl.kernel,
    out_type=jax.ShapeDtypeStruct((B,), jnp.float32),
    mesh=_mesh,
    compiler_params=pltpu.CompilerParams(needs_layout_passes=False),
    scratch_types=[
        pltpu.VMEM((BPW,), jnp.int32),
        pltpu.VMEM((BPW,), jnp.int32),
        pltpu.VMEM((G, F, 128), jnp.float32),
        pltpu.VMEM((G, F, 128), jnp.float32),
        pltpu.VMEM((BPW,), jnp.float32),
        pltpu.SemaphoreType.DMA,
        pltpu.SemaphoreType.DMA,
    ],
)
def _mf_score(task_hbm, worker_hbm, tfT_hbm, wfT_hbm, out_hbm,
              t_idx, w_idx, t_slab, w_slab, out_v, sem_idx, sem):
    wid = lax.axis_index("s") * NC + lax.axis_index("c")
    base = wid * BPW

    cp_t = pltpu.async_copy(task_hbm.at[pl.ds(base, BPW)], t_idx, sem_idx)
    cp_w = pltpu.async_copy(worker_hbm.at[pl.ds(base, BPW)], w_idx, sem_idx)
    cp_t.wait()
    cp_w.wait()

    lanes = lax.iota(jnp.int32, L)

    def step(g, carry):
        off = pl.multiple_of(g * G, G)
        ivt = t_idx[pl.ds(off, G)]
        ivw = w_idx[pl.ds(off, G)]
        cps = []
        for k in range(G):
            ct = pl.multiple_of(ivt[k] & -128, 128)
            cw = pl.multiple_of(ivw[k] & -128, 128)
            cps.append(pltpu.async_copy(
                tfT_hbm.at[pl.ds(0, F), pl.ds(ct, 128)], t_slab.at[k], sem))
            cps.append(pltpu.async_copy(
                wfT_hbm.at[pl.ds(0, F), pl.ds(cw, 128)], w_slab.at[k], sem))
        for cp in cps:
            cp.wait()

        lt = ivt & 127
        lw = ivw & 127
        acc = jnp.zeros((L,), jnp.float32)
        for f in range(F):
            fv = jnp.full((L,), f, jnp.int32)
            tv = plsc.load_gather(t_slab, [lanes, fv, lt])
            wv = plsc.load_gather(w_slab, [lanes, fv, lw])
            acc = acc + tv * wv
        out_v[pl.ds(off, G)] = 1.0 / (1.0 + jnp.exp(-acc))
        return carry

    lax.fori_loop(0, NSTEP, step, 0)

    pltpu.sync_copy(out_v, out_hbm.at[pl.ds(base, BPW)])


def kernel(task, worker, task_factors, worker_factors):
    return _mf_score(task.astype(jnp.int32), worker.astype(jnp.int32),
                     task_factors.T, worker_factors.T)


# speedup vs baseline: 6.1572x; 1.0054x over previous
import functools

import jax
import jax.numpy as jnp
from jax import lax
from jax.experimental import pallas as pl
from jax.experimental.pallas import tpu as pltpu
from jax.experimental.pallas import tpu_sc as plsc

NC = 2
NS = 16
L = 16
NW = NC * NS

B = 16384
F = 16
BPW = B // NW
G = 16
NSTEP = BPW // G

_mesh = plsc.VectorSubcoreMesh(core_axis_name="c", subcore_axis_name="s")


@functools.partial(
    pl.kernel,
    out_type=jax.ShapeDtypeStruct((B,), jnp.float32),
    mesh=_mesh,
    compiler_params=pltpu.CompilerParams(
        needs_layout_passes=False,
        disable_bounds_checks=True,
        disable_semaphore_checks=True,
    ),
    scratch_types=[
        pltpu.VMEM((BPW,), jnp.int32),
        pltpu.VMEM((BPW,), jnp.int32),
        pltpu.VMEM((G, F, 128), jnp.float32),
        pltpu.VMEM((G, F, 128), jnp.float32),
        pltpu.VMEM((BPW,), jnp.float32),
        pltpu.SemaphoreType.DMA,
        pltpu.SemaphoreType.DMA,
    ],
)
def _mf_score(task_hbm, worker_hbm, tfT_hbm, wfT_hbm, out_hbm,
              t_idx, w_idx, t_slab, w_slab, out_v, sem_idx, sem):
    wid = lax.axis_index("s") * NC + lax.axis_index("c")
    base = wid * BPW

    cp_t = pltpu.async_copy(task_hbm.at[pl.ds(base, BPW)], t_idx, sem_idx)
    cp_w = pltpu.async_copy(worker_hbm.at[pl.ds(base, BPW)], w_idx, sem_idx)
    cp_t.wait()
    cp_w.wait()

    lanes = lax.iota(jnp.int32, L)

    def step(g, carry):
        off = pl.multiple_of(g * G, G)
        ivt = t_idx[pl.ds(off, G)]
        ivw = w_idx[pl.ds(off, G)]
        cps = []
        for k in range(G):
            ct = pl.multiple_of(ivt[k] & -128, 128)
            cw = pl.multiple_of(ivw[k] & -128, 128)
            cps.append(pltpu.async_copy(
                tfT_hbm.at[pl.ds(0, F), pl.ds(ct, 128)], t_slab.at[k], sem))
            cps.append(pltpu.async_copy(
                wfT_hbm.at[pl.ds(0, F), pl.ds(cw, 128)], w_slab.at[k], sem))
        for cp in cps:
            cp.wait()

        lt = ivt & 127
        lw = ivw & 127
        acc = jnp.zeros((L,), jnp.float32)
        for f in range(F):
            fv = jnp.full((L,), f, jnp.int32)
            tv = plsc.load_gather(t_slab, [lanes, fv, lt])
            wv = plsc.load_gather(w_slab, [lanes, fv, lw])
            acc = acc + tv * wv
        out_v[pl.ds(off, G)] = 1.0 / (1.0 + jnp.exp(-acc))
        return carry

    lax.fori_loop(0, NSTEP, step, 0)

    pltpu.sync_copy(out_v, out_hbm.at[pl.ds(base, BPW)])


def kernel(task, worker, task_factors, worker_factors):
    return _mf_score(task.astype(jnp.int32), worker.astype(jnp.int32),
                     task_factors.T, worker_factors.T)


# speedup vs baseline: 6.1642x; 1.0011x over previous
import functools

import jax
import jax.numpy as jnp
from jax import lax
from jax.experimental import pallas as pl
from jax.experimental.pallas import tpu as pltpu
from jax.experimental.pallas import tpu_sc as plsc

NC = 2
NS = 16
L = 16
NW = NC * NS

B = 16384
F = 16
BPW = B // NW
G = 16
NSTEP = BPW // G

_mesh = plsc.VectorSubcoreMesh(core_axis_name="c", subcore_axis_name="s")


@functools.partial(
    pl.kernel,
    out_type=jax.ShapeDtypeStruct((B,), jnp.float32),
    mesh=_mesh,
    compiler_params=pltpu.CompilerParams(
        needs_layout_passes=False,
        disable_bounds_checks=True,
        disable_semaphore_checks=True,
        skip_device_barrier=True,
    ),
    scratch_types=[
        pltpu.VMEM((BPW,), jnp.int32),
        pltpu.VMEM((BPW,), jnp.int32),
        pltpu.VMEM((G, F, 128), jnp.float32),
        pltpu.VMEM((G, F, 128), jnp.float32),
        pltpu.VMEM((BPW,), jnp.float32),
        pltpu.SemaphoreType.DMA,
        pltpu.SemaphoreType.DMA,
    ],
)
def _mf_score(task_hbm, worker_hbm, tfT_hbm, wfT_hbm, out_hbm,
              t_idx, w_idx, t_slab, w_slab, out_v, sem_idx, sem):
    wid = lax.axis_index("s") * NC + lax.axis_index("c")
    base = wid * BPW

    cp_t = pltpu.async_copy(task_hbm.at[pl.ds(base, BPW)], t_idx, sem_idx)
    cp_w = pltpu.async_copy(worker_hbm.at[pl.ds(base, BPW)], w_idx, sem_idx)
    cp_t.wait()
    cp_w.wait()

    lanes = lax.iota(jnp.int32, L)

    def step(g, carry):
        off = pl.multiple_of(g * G, G)
        ivt = t_idx[pl.ds(off, G)]
        ivw = w_idx[pl.ds(off, G)]
        cps = []
        for k in range(G):
            ct = pl.multiple_of(ivt[k] & -128, 128)
            cw = pl.multiple_of(ivw[k] & -128, 128)
            cps.append(pltpu.async_copy(
                tfT_hbm.at[pl.ds(0, F), pl.ds(ct, 128)], t_slab.at[k], sem))
            cps.append(pltpu.async_copy(
                wfT_hbm.at[pl.ds(0, F), pl.ds(cw, 128)], w_slab.at[k], sem))
        for cp in cps:
            cp.wait()

        lt = ivt & 127
        lw = ivw & 127
        acc = jnp.zeros((L,), jnp.float32)
        for f in range(F):
            fv = jnp.full((L,), f, jnp.int32)
            tv = plsc.load_gather(t_slab, [lanes, fv, lt])
            wv = plsc.load_gather(w_slab, [lanes, fv, lw])
            acc = acc + tv * wv
        out_v[pl.ds(off, G)] = 1.0 / (1.0 + jnp.exp(-acc))
        return carry

    lax.fori_loop(0, NSTEP, step, 0)

    pltpu.sync_copy(out_v, out_hbm.at[pl.ds(base, BPW)])


def kernel(task, worker, task_factors, worker_factors):
    return _mf_score(task.astype(jnp.int32), worker.astype(jnp.int32),
                     task_factors.T, worker_factors.T)


# speedup vs baseline: 6.2072x; 1.0070x over previous
import functools

import jax
import jax.numpy as jnp
from jax import lax
from jax.experimental import pallas as pl
from jax.experimental.pallas import tpu as pltpu
from jax.experimental.pallas import tpu_sc as plsc

NC = 2
NS = 16
L = 16
NW = NC * NS

B = 16384
F = 16
H = F // 2
BPW = B // NW
G = 16
NSTEP = BPW // G

_mesh = plsc.VectorSubcoreMesh(core_axis_name="c", subcore_axis_name="s")


@functools.partial(
    pl.kernel,
    out_type=jax.ShapeDtypeStruct((B,), jnp.float32),
    mesh=_mesh,
    compiler_params=pltpu.CompilerParams(
        needs_layout_passes=False,
        disable_bounds_checks=True,
        disable_semaphore_checks=True,
        skip_device_barrier=True,
    ),
    scratch_types=[
        pltpu.VMEM((BPW,), jnp.int32),
        pltpu.VMEM((BPW,), jnp.int32),
        pltpu.VMEM((G, H, 128), jnp.float32),
        pltpu.VMEM((G, H, 128), jnp.float32),
        pltpu.VMEM((G, H, 128), jnp.float32),
        pltpu.VMEM((G, H, 128), jnp.float32),
        pltpu.VMEM((BPW,), jnp.float32),
        pltpu.SemaphoreType.DMA,
        pltpu.SemaphoreType.DMA,
        pltpu.SemaphoreType.DMA,
    ],
)
def _mf_score(task_hbm, worker_hbm, tfT_hbm, wfT_hbm, out_hbm,
              t_idx, w_idx, a_t, a_w, b_t, b_w, out_v,
              sem_idx, sem_a, sem_b):
    wid = lax.axis_index("s") * NC + lax.axis_index("c")
    base = wid * BPW

    cp_t = pltpu.async_copy(task_hbm.at[pl.ds(base, BPW)], t_idx, sem_idx)
    cp_w = pltpu.async_copy(worker_hbm.at[pl.ds(base, BPW)], w_idx, sem_idx)
    cp_t.wait()
    cp_w.wait()

    lanes = lax.iota(jnp.int32, L)

    def load_cols(g):
        off = pl.multiple_of(g * G, G)
        return t_idx[pl.ds(off, G)], w_idx[pl.ds(off, G)]

    def fire_half(ivt, ivw, rows, dst_t, dst_w, sem):
        for k in range(G):
            ct = pl.multiple_of(ivt[k] & -128, 128)
            cw = pl.multiple_of(ivw[k] & -128, 128)
            pltpu.async_copy(
                tfT_hbm.at[rows, pl.ds(ct, 128)], dst_t.at[k], sem)
            pltpu.async_copy(
                wfT_hbm.at[rows, pl.ds(cw, 128)], dst_w.at[k], sem)

    def drain(dst_t, dst_w, sem):
        dummy = tfT_hbm.at[pl.ds(0, H), pl.ds(0, 128)]
        for k in range(G):
            pltpu.make_async_copy(dummy, dst_t.at[k], sem).wait()
            pltpu.make_async_copy(dummy, dst_w.at[k], sem).wait()

    def half_dot(src_t, src_w, lt, lw):
        acc = jnp.zeros((L,), jnp.float32)
        for f in range(H):
            fv = jnp.full((L,), f, jnp.int32)
            tv = plsc.load_gather(src_t, [lanes, fv, lt])
            wv = plsc.load_gather(src_w, [lanes, fv, lw])
            acc = acc + tv * wv
        return acc

    rows0 = pl.ds(0, H)
    rows1 = pl.ds(H, H)

    ivt0, ivw0 = load_cols(0)
    fire_half(ivt0, ivw0, rows0, a_t, a_w, sem_a)

    def step(g, carry):
        ivt, ivw = load_cols(g)
        lt = ivt & 127
        lw = ivw & 127
        fire_half(ivt, ivw, rows1, b_t, b_w, sem_b)
        drain(a_t, a_w, sem_a)
        c0 = half_dot(a_t, a_w, lt, lw)

        @pl.when(g + 1 < NSTEP)
        def _():
            nivt, nivw = load_cols(g + 1)
            fire_half(nivt, nivw, rows0, a_t, a_w, sem_a)

        drain(b_t, b_w, sem_b)
        c1 = half_dot(b_t, b_w, lt, lw)

        off = pl.multiple_of(g * G, G)
        out_v[pl.ds(off, G)] = 1.0 / (1.0 + jnp.exp(-(c0 + c1)))
        return carry

    lax.fori_loop(0, NSTEP, step, 0)

    pltpu.sync_copy(out_v, out_hbm.at[pl.ds(base, BPW)])


def kernel(task, worker, task_factors, worker_factors):
    return _mf_score(task.astype(jnp.int32), worker.astype(jnp.int32),
                     task_factors.T, worker_factors.T)
